# baseline (speedup 1.0000x reference)
import functools

import jax
import jax.numpy as jnp
from jax import lax
from jax.experimental import pallas as pl
from jax.experimental.pallas import tpu as pltpu
from jax.experimental.pallas import tpu_sc as plsc

B, S, D = 4, 2048, 1024
HALF = D // 2
T = B * S
NC, NS = 2, 16
NW = NC * NS
PER_W = T // NW
CHUNK = 32
NCHUNK = PER_W // CHUNK
LANES = 16
VPH = HALF // LANES

_mesh = plsc.VectorSubcoreMesh(
    core_axis_name="c", subcore_axis_name="s", num_cores=NC, num_subcores=NS)


@functools.partial(
    pl.kernel,
    out_type=jax.ShapeDtypeStruct((T, D), jnp.float32),
    mesh=_mesh,
    scratch_types=[
        pltpu.VMEM((PER_W,), jnp.int32),
        pltpu.VMEM((PER_W,), jnp.int32),
        pltpu.VMEM((CHUNK, D), jnp.float32),
        pltpu.VMEM((CHUNK, HALF), jnp.float32),
        pltpu.VMEM((CHUNK, HALF), jnp.float32),
        pltpu.SemaphoreType.DMA,
    ],
)
def _pos_emb_add(x_hbm, idx0_hbm, idx1_hbm, pe1_hbm, pe2_hbm, out_hbm,
                 idx0_v, idx1_v, x_v, g1_v, g2_v, sem):
    wid = lax.axis_index("s") * NC + lax.axis_index("c")
    base = wid * PER_W
    pltpu.sync_copy(idx0_hbm.at[pl.ds(base, PER_W)], idx0_v)
    pltpu.sync_copy(idx1_hbm.at[pl.ds(base, PER_W)], idx1_v)
    for c in range(NCHUNK):
        off = base + c * CHUNK
        cp_x = pltpu.async_copy(x_hbm.at[pl.ds(off, CHUNK)], x_v, sem)
        cp_g1 = pltpu.async_copy(
            pe1_hbm.at[idx0_v.at[pl.ds(c * CHUNK, CHUNK)]], g1_v, sem)
        cp_g2 = pltpu.async_copy(
            pe2_hbm.at[idx1_v.at[pl.ds(c * CHUNK, CHUNK)]], g2_v, sem)
        cp_x.wait()
        cp_g1.wait()
        cp_g2.wait()

        def add_row(k, _):
            for j in range(VPH):
                plsc.addupdate(x_v.at[k, pl.ds(j * LANES, LANES)],
                               g1_v[k, pl.ds(j * LANES, LANES)])
                plsc.addupdate(x_v.at[k, pl.ds(HALF + j * LANES, LANES)],
                               g2_v[k, pl.ds(j * LANES, LANES)])
            return _

        lax.fori_loop(0, CHUNK, add_row, 0)
        pltpu.sync_copy(x_v, out_hbm.at[pl.ds(off, CHUNK)])


def kernel(inputs, positions, pe1, pe2):
    pos = positions.astype(jnp.int32).reshape(T, 2)
    out = _pos_emb_add(inputs.reshape(T, D), pos[:, 0], pos[:, 1], pe1, pe2)
    return out.reshape(B, S, D)
